# baseline (speedup 1.0000x reference)
import functools

import jax
import jax.numpy as jnp
from jax import lax
from jax.experimental import pallas as pl
from jax.experimental.pallas import tpu as pltpu
from jax.experimental.pallas import tpu_sc as plsc

_BATCH, _HIST, _DIM = 4096, 200, 32
_B = _BATCH * _HIST

_info = plsc.get_sparse_core_info()
_NC, _NS = _info.num_cores, _info.num_subcores
_NW = _NC * _NS
_BT = _BATCH // _NW
_GR = _DIM // 8

_mesh = plsc.VectorSubcoreMesh(core_axis_name="c", subcore_axis_name="s")


@functools.partial(
    pl.kernel,
    mesh=_mesh,
    out_type=jax.ShapeDtypeStruct((_HIST, _GR, _NW, 8, 128), jnp.float32),
    scratch_types=[
        pltpu.VMEM((_BT * _HIST,), jnp.int32),
        pltpu.VMEM((_BT,), jnp.int32),
        pltpu.VMEM((_BT,), jnp.int32),
        pltpu.VMEM((_BT, _DIM), jnp.float32),
        pltpu.VMEM((_BT, _DIM), jnp.float32),
        pltpu.VMEM((_GR, 8, 128), jnp.float32),
        pltpu.VMEM((_GR, 8, 128), jnp.float32),
        pltpu.SemaphoreType.DMA,
        pltpu.SemaphoreType.DMA,
    ],
    compiler_params=pltpu.CompilerParams(
        use_tc_tiling_on_sc=False, needs_layout_passes=False
    ),
)
def _gather(idx_hbm, table_hbm, out_hbm, idx_all, ih0, ih1, r0, r1, t0, t1,
            gat_sem, out_sem):
    w = lax.axis_index("s") * _NC + lax.axis_index("c")
    idx_h = [ih0, ih1]
    rows = [r0, r1]
    trans = [t0, t1]

    lane = jnp.arange(16, dtype=jnp.int32)
    lane_h = lane * _HIST

    def build_idx(h, b):
        for j in range(_BT // 16):
            v = plsc.load_gather(idx_all, [lane_h + (j * 16 * _HIST + h)])
            idx_h[b][pl.ds(j * 16, 16)] = v

    def fire_gather(b):
        pltpu.async_copy(table_hbm.at[idx_h[b]], rows[b], gat_sem)

    def wait_gather(b):
        pltpu.make_async_copy(table_hbm.at[idx_h[b]], rows[b], gat_sem).wait()

    def transpose(b):
        for g in range(_GR):
            for dr in range(8):
                d = g * 8 + dr
                col = jnp.full((16,), d, dtype=jnp.int32)
                for j in range(_BT // 16):
                    v = plsc.load_gather(rows[b], [lane + j * 16, col])
                    trans[b][g, dr, pl.ds(j * 16, 16)] = v

    def fire_outs(h, b):
        for g in range(_GR):
            pltpu.async_copy(trans[b].at[g], out_hbm.at[h, g, w], out_sem)

    def wait_outs(h, b):
        for g in range(_GR):
            pltpu.make_async_copy(
                trans[b].at[g], out_hbm.at[h, g, w], out_sem
            ).wait()

    pltpu.sync_copy(idx_hbm.at[pl.ds(w * _BT * _HIST, _BT * _HIST)], idx_all)
    build_idx(0, 0)
    fire_gather(0)

    def body(h2, carry):
        for p in range(2):
            h = h2 * 2 + p
            b = p
            nh = h + 1
            @pl.when(nh < _HIST)
            def _():
                build_idx(nh, 1 - b)
                fire_gather(1 - b)

            wait_gather(b)

            @pl.when(h >= 2)
            def _():
                wait_outs(h - 2, b)

            transpose(b)
            fire_outs(h, b)
        return carry

    lax.fori_loop(0, _HIST // 2, body, 0)
    wait_outs(_HIST - 2, 0)
    wait_outs(_HIST - 1, 1)


def kernel(inputs, table):
    flat = inputs.reshape(-1).astype(jnp.int32)
    five = _gather(flat, table)
    return five.transpose(2, 4, 0, 1, 3).reshape(_BATCH, _HIST, _DIM)


# speedup vs baseline: 1.1586x; 1.1586x over previous
import functools

import jax
import jax.numpy as jnp
from jax import lax
from jax.experimental import pallas as pl
from jax.experimental.pallas import tpu as pltpu
from jax.experimental.pallas import tpu_sc as plsc

_BATCH, _HIST, _DIM = 4096, 200, 32
_B = _BATCH * _HIST

_info = plsc.get_sparse_core_info()
_NC, _NS = _info.num_cores, _info.num_subcores
_NW = _NC * _NS
_BT = _BATCH // _NW
_GR = _DIM // 8

_mesh = plsc.VectorSubcoreMesh(core_axis_name="c", subcore_axis_name="s")


@functools.partial(
    pl.kernel,
    mesh=_mesh,
    out_type=jax.ShapeDtypeStruct((_HIST, _GR, _NW, 1024), jnp.float32),
    scratch_types=[
        pltpu.VMEM((_BT * _HIST,), jnp.int32),
        pltpu.VMEM((_BT,), jnp.int32),
        pltpu.VMEM((_BT,), jnp.int32),
        pltpu.VMEM((_BT, _DIM), jnp.float32),
        pltpu.VMEM((_BT, _DIM), jnp.float32),
        pltpu.VMEM((_GR * 8 * 128,), jnp.float32),
        pltpu.VMEM((_GR * 8 * 128,), jnp.float32),
        pltpu.SemaphoreType.DMA,
        pltpu.SemaphoreType.DMA,
    ],
    compiler_params=pltpu.CompilerParams(
        use_tc_tiling_on_sc=False, needs_layout_passes=False
    ),
)
def _gather(idx_hbm, table_hbm, out_hbm, idx_all, ih0, ih1, r0, r1, t0, t1,
            gat_sem, out_sem):
    w = lax.axis_index("s") * _NC + lax.axis_index("c")
    idx_h = [ih0, ih1]
    rows = [r0, r1]
    trans = [t0, t1]

    lane = jnp.arange(16, dtype=jnp.int32)
    lane_h = lane * _HIST

    def build_idx(h, b):
        for j in range(_BT // 16):
            v = plsc.load_gather(idx_all, [lane_h + (j * 16 * _HIST + h)])
            idx_h[b][pl.ds(j * 16, 16)] = v

    def fire_gather(b):
        pltpu.async_copy(table_hbm.at[idx_h[b]], rows[b], gat_sem)

    def wait_gather(b):
        pltpu.make_async_copy(table_hbm.at[idx_h[b]], rows[b], gat_sem).wait()

    lane_lo = lane * 128
    lane_hi = (lane + 16) * 128

    def transpose(b):
        for bc in range(_BT):
            v_lo = rows[b][bc, pl.ds(0, 16)]
            v_hi = rows[b][bc, pl.ds(16, 16)]
            plsc.store_scatter(trans[b], [lane_lo + bc], v_lo)
            plsc.store_scatter(trans[b], [lane_hi + bc], v_hi)

    def fire_outs(h, b):
        for g in range(_GR):
            pltpu.async_copy(
                trans[b].at[pl.ds(g * 1024, 1024)], out_hbm.at[h, g, w], out_sem
            )

    def wait_outs(h, b):
        for g in range(_GR):
            pltpu.make_async_copy(
                trans[b].at[pl.ds(g * 1024, 1024)], out_hbm.at[h, g, w], out_sem
            ).wait()

    pltpu.sync_copy(idx_hbm.at[pl.ds(w * _BT * _HIST, _BT * _HIST)], idx_all)
    build_idx(0, 0)
    fire_gather(0)

    def body(h2, carry):
        for p in range(2):
            h = h2 * 2 + p
            b = p
            nh = h + 1
            @pl.when(nh < _HIST)
            def _():
                build_idx(nh, 1 - b)
                fire_gather(1 - b)

            wait_gather(b)

            @pl.when(h >= 2)
            def _():
                wait_outs(h - 2, b)

            transpose(b)
            fire_outs(h, b)
        return carry

    lax.fori_loop(0, _HIST // 2, body, 0)
    wait_outs(_HIST - 2, 0)
    wait_outs(_HIST - 1, 1)


def kernel(inputs, table):
    flat = inputs.reshape(-1).astype(jnp.int32)
    four = _gather(flat, table)
    five = four.reshape(_HIST, _GR, _NW, 8, 128)
    return five.transpose(2, 4, 0, 1, 3).reshape(_BATCH, _HIST, _DIM)


# speedup vs baseline: 1.1689x; 1.0089x over previous
import functools

import jax
import jax.numpy as jnp
from jax import lax
from jax.experimental import pallas as pl
from jax.experimental.pallas import tpu as pltpu
from jax.experimental.pallas import tpu_sc as plsc

_BATCH, _HIST, _DIM = 4096, 200, 32
_B = _BATCH * _HIST

_info = plsc.get_sparse_core_info()
_NC, _NS = _info.num_cores, _info.num_subcores
_NW = _NC * _NS
_BPW = _B // _NW
_CH = 1600
_NCHUNK = _BPW // _CH

_mesh = plsc.VectorSubcoreMesh(core_axis_name="c", subcore_axis_name="s")


@functools.partial(
    pl.kernel,
    mesh=_mesh,
    out_type=jax.ShapeDtypeStruct((_B, _DIM), jnp.float32),
    scratch_types=[
        pltpu.VMEM((_CH,), jnp.int32),
        pltpu.VMEM((_CH,), jnp.int32),
        pltpu.VMEM((_CH, _DIM), jnp.float32),
        pltpu.VMEM((_CH, _DIM), jnp.float32),
        pltpu.SemaphoreType.DMA,
        pltpu.SemaphoreType.DMA,
    ],
    compiler_params=pltpu.CompilerParams(use_tc_tiling_on_sc=False),
)
def _gather(idx_hbm, table_hbm, out_hbm, idx0, idx1, rows0, rows1, gat_sem, out_sem):
    wid = lax.axis_index("s") * _NC + lax.axis_index("c")
    base = wid * _BPW
    idx_v = [idx0, idx1]
    rows_v = [rows0, rows1]

    def load_idx(i, b):
        pltpu.sync_copy(idx_hbm.at[pl.ds(base + i * _CH, _CH)], idx_v[b])

    def start_gather(b):
        return pltpu.async_copy(table_hbm.at[idx_v[b]], rows_v[b], gat_sem)

    def start_out(i, b):
        return pltpu.async_copy(
            rows_v[b], out_hbm.at[pl.ds(base + i * _CH, _CH)], out_sem
        )

    load_idx(0, 0)
    gathers = [start_gather(0)]
    outs = []
    for i in range(_NCHUNK):
        b = i % 2
        if i + 1 < _NCHUNK:
            load_idx(i + 1, 1 - b)
        gathers[i].wait()
        if i >= 1:
            outs[i - 1].wait()
        if i + 1 < _NCHUNK:
            gathers.append(start_gather(1 - b))
        outs.append(start_out(i, b))
    outs[-1].wait()


def kernel(inputs, table):
    flat = inputs.reshape(-1).astype(jnp.int32)
    out = _gather(flat, table)
    return out.reshape(_BATCH, _HIST, _DIM)
